# baseline (speedup 1.0000x reference)
import functools

import jax
import jax.numpy as jnp
from jax import lax
from jax.experimental import pallas as pl
from jax.experimental.pallas import tpu as pltpu
from jax.experimental.pallas import tpu_sc as plsc

N = 10000
D = 128
E = 320000

NC = 2
NS = 16
NW = NC * NS
CHUNK = 128
C = 80
EPAD = NW * C * CHUNK
DUMMY = N
NPAD = 10240
RPT = NPAD // NS

_BN_SCALE = 1.0 / (1.0 + 1e-5) ** 0.5
_LN_EPS = 1e-5


def _zero_vmem_rows(buf, nrows, ncols16):
    def zrow(i, _):
        for k in range(ncols16):
            buf[i, pl.ds(k * 16, 16)] = jnp.zeros((16,), jnp.float32)
        return 0
    lax.fori_loop(0, nrows, zrow, 0)


def _deg_body(dst_hbm, out_hbm, dst_v, buf, acc_sh):
    c = lax.axis_index("c")
    s = lax.axis_index("s")
    pltpu.sync_copy(dst_hbm.at[c, s], dst_v)
    _zero_vmem_rows(buf, CHUNK, D // 16)
    for b in range(RPT // CHUNK):
        pltpu.sync_copy(buf, acc_sh.at[pl.ds(s * RPT + b * CHUNK, CHUNK)])
    def orow(i, _):
        for k in range(D // 16):
            buf[i, pl.ds(k * 16, 16)] = jnp.ones((16,), jnp.float32)
        return 0
    lax.fori_loop(0, CHUNK, orow, 0)
    plsc.subcore_barrier()
    def body(j, _):
        pltpu.sync_copy(buf, acc_sh.at[dst_v.at[j]], add=True)
        return 0
    lax.fori_loop(0, C, body, 0)
    plsc.subcore_barrier()
    pltpu.sync_copy(acc_sh.at[pl.ds(s * RPT, RPT)],
                    out_hbm.at[c, pl.ds(s * RPT, RPT)])


def _sc_degree(dst_idx):
    mesh = plsc.VectorSubcoreMesh(core_axis_name="c", subcore_axis_name="s")
    return pl.kernel(
        _deg_body,
        out_type=jax.ShapeDtypeStruct((NC, NPAD, D), jnp.float32),
        mesh=mesh,
        scratch_types=[
            pltpu.VMEM((C, CHUNK), jnp.int32),
            pltpu.VMEM((CHUNK, D), jnp.float32),
            pltpu.VMEM_SHARED((NPAD, D), jnp.float32),
        ],
    )(dst_idx)


NBUF = 2
G = 16
NBLK = C // G
GB = CHUNK * D * 4


def _prop_body(g_hbm, src_hbm, dst_hbm, out_hbm, isrc, idst, gbuf, acc_sh,
               gs0, gs1, ss0, ss1):
    gsem = (gs0, gs1)
    ssem = (ss0, ss1)
    c = lax.axis_index("c")
    s = lax.axis_index("s")
    def zrow(i, _):
        for k in range(D // 16):
            gbuf[0, i, pl.ds(k * 16, 16)] = jnp.zeros((16,), jnp.float32)
        return 0
    lax.fori_loop(0, CHUNK, zrow, 0)
    for b in range(RPT // CHUNK):
        pltpu.sync_copy(gbuf.at[0], acc_sh.at[pl.ds(s * RPT + b * CHUNK,
                                                    CHUNK)])
    plsc.subcore_barrier()

    def load_blk(k, slot):
        pltpu.sync_copy(src_hbm.at[c, s, pl.ds(k * G, G)], isrc.at[slot])
        pltpu.sync_copy(dst_hbm.at[c, s, pl.ds(k * G, G)], idst.at[slot])

    def gather(slot, j, b):
        pltpu.async_copy(g_hbm.at[isrc.at[slot, j]], gbuf.at[b], gsem[b])

    def scatter(slot, j, b):
        pltpu.async_copy(gbuf.at[b], acc_sh.at[idst.at[slot, j]], ssem[b],
                         add=True)

    def gwait(b):
        pltpu.make_async_copy(g_hbm.at[isrc.at[0, 0]], gbuf.at[b],
                              gsem[b]).wait()

    def swait(b):
        pltpu.make_async_copy(gbuf.at[b], acc_sh.at[idst.at[0, 0]],
                              ssem[b]).wait()

    load_blk(0, 0)

    def block(k, slot):
        gather(slot, 0, 0)

        @pl.when(k < NBLK - 1)
        def _():
            load_blk(k + 1, 1 - slot)

        for j in range(G):
            b = j % NBUF
            gwait(b)
            scatter(slot, j, b)
            if j + 1 < G:
                if j >= 1:
                    swait(1 - b)
                gather(slot, j + 1, 1 - b)
        swait((G - 1) % NBUF)
        swait((G - 2) % NBUF)
        return 1 - slot

    slot = 0
    def bstep(k, slot_):
        return block(k, slot_)
    lax.fori_loop(0, NBLK, bstep, slot)

    plsc.subcore_barrier()
    pltpu.sync_copy(acc_sh.at[pl.ds(s * RPT, RPT)],
                    out_hbm.at[c, pl.ds(s * RPT, RPT)])


def _sc_propagate(g, src_idx, dst_idx):
    mesh = plsc.VectorSubcoreMesh(core_axis_name="c", subcore_axis_name="s")
    return pl.kernel(
        _prop_body,
        out_type=jax.ShapeDtypeStruct((NC, NPAD, D), jnp.float32),
        mesh=mesh,
        scratch_types=[
            pltpu.VMEM((2, G, CHUNK), jnp.int32),
            pltpu.VMEM((2, G, CHUNK), jnp.int32),
            pltpu.VMEM((NBUF, CHUNK, D), jnp.float32),
            pltpu.VMEM_SHARED((NPAD, D), jnp.float32),
        ] + [pltpu.SemaphoreType.DMA] * 4,
    )(g, src_idx, dst_idx)


BLK = 1024
BLKP = 1000


def _pre_body(x_ref, w_ref, d0_ref, d1_ref, g_ref, dinv_ref):
    deg = d0_ref[:, 0:1] + d1_ref[:, 0:1] + 1.0
    dinv = lax.rsqrt(deg)
    g_ref[...] = jnp.dot(x_ref[...], w_ref[...],
                         preferred_element_type=jnp.float32) * dinv
    dinv_ref[...] = jnp.broadcast_to(dinv, (BLK, 16))


def _tc_pre(xp, W1, deg0, deg1):
    grid = NPAD // BLK
    return pl.pallas_call(
        _pre_body,
        grid=(grid,),
        in_specs=[
            pl.BlockSpec((BLK, D), lambda i: (i, 0)),
            pl.BlockSpec((D, D), lambda i: (0, 0)),
            pl.BlockSpec((BLK, D), lambda i: (i, 0)),
            pl.BlockSpec((BLK, D), lambda i: (i, 0)),
        ],
        out_specs=[
            pl.BlockSpec((BLK, D), lambda i: (i, 0)),
            pl.BlockSpec((BLK, 16), lambda i: (i, 0)),
        ],
        out_shape=[
            jax.ShapeDtypeStruct((NPAD, D), jnp.float32),
            jax.ShapeDtypeStruct((NPAD, 16), jnp.float32),
        ],
    )(xp, W1, deg0, deg1)


def _mid_body(p0_ref, p1_ref, g_ref, dinv_ref, b_ref, bng_ref, bnb_ref, w_ref,
              out_ref):
    dv = dinv_ref[:, 0:1]
    pre = (p0_ref[...] + p1_ref[...] + g_ref[...]) * dv + b_ref[...]
    h = jnp.maximum(pre * _BN_SCALE * bng_ref[...] + bnb_ref[...], 0.0)
    out_ref[...] = jnp.dot(h, w_ref[...],
                           preferred_element_type=jnp.float32) * dv


def _tc_mid(p0, p1, g, dinv16, b, bng, bnb, Wn):
    grid = NPAD // BLK
    return pl.pallas_call(
        _mid_body,
        grid=(grid,),
        in_specs=[
            pl.BlockSpec((BLK, D), lambda i: (i, 0)),
            pl.BlockSpec((BLK, D), lambda i: (i, 0)),
            pl.BlockSpec((BLK, D), lambda i: (i, 0)),
            pl.BlockSpec((BLK, 16), lambda i: (i, 0)),
            pl.BlockSpec((1, D), lambda i: (0, 0)),
            pl.BlockSpec((1, D), lambda i: (0, 0)),
            pl.BlockSpec((1, D), lambda i: (0, 0)),
            pl.BlockSpec((D, D), lambda i: (0, 0)),
        ],
        out_specs=pl.BlockSpec((BLK, D), lambda i: (i, 0)),
        out_shape=jax.ShapeDtypeStruct((NPAD, D), jnp.float32),
    )(p0, p1, g, dinv16, b, bng, bnb, Wn)


def _post_body(p0_ref, p1_ref, g_ref, dinv_ref, b_ref, lng_ref, lnb_ref,
               emb_ref, pool_ref):
    i = pl.program_id(0)
    ngrid = pl.num_programs(0)
    dv = dinv_ref[:, 0:1]
    h = (p0_ref[...] + p1_ref[...] + g_ref[...]) * dv + b_ref[...]
    mu = jnp.mean(h, axis=-1, keepdims=True)
    var = jnp.mean((h - mu) ** 2, axis=-1, keepdims=True)
    e = (h - mu) * lax.rsqrt(var + _LN_EPS) * lng_ref[...] + lnb_ref[...]
    emb_ref[...] = e
    bsum = jnp.sum(e, axis=0, keepdims=True)

    @pl.when(i == 0)
    def _():
        pool_ref[...] = jnp.zeros_like(pool_ref)

    pool_ref[...] += bsum

    @pl.when(i == ngrid - 1)
    def _():
        pool_ref[...] = pool_ref[...] * (1.0 / N)


def _tc_post(p0, p1, g, dinv16, b, lng, lnb):
    grid = N // BLKP
    return pl.pallas_call(
        _post_body,
        grid=(grid,),
        in_specs=[
            pl.BlockSpec((BLKP, D), lambda i: (i, 0)),
            pl.BlockSpec((BLKP, D), lambda i: (i, 0)),
            pl.BlockSpec((BLKP, D), lambda i: (i, 0)),
            pl.BlockSpec((BLKP, 16), lambda i: (i, 0)),
            pl.BlockSpec((1, D), lambda i: (0, 0)),
            pl.BlockSpec((1, D), lambda i: (0, 0)),
            pl.BlockSpec((1, D), lambda i: (0, 0)),
        ],
        out_specs=[
            pl.BlockSpec((BLKP, D), lambda i: (i, 0)),
            pl.BlockSpec((1, D), lambda i: (0, 0)),
        ],
        out_shape=[
            jax.ShapeDtypeStruct((N, D), jnp.float32),
            jax.ShapeDtypeStruct((1, D), jnp.float32),
        ],
    )(p0, p1, g, dinv16, b, lng, lnb)


def kernel(x, edge_index, W1, b1, W2, b2, W3, b3, bn1_g, bn1_b, bn2_g, bn2_b,
           ln_g, ln_b):
    xp = jnp.zeros((NPAD, D), jnp.float32).at[:N].set(x)
    pad = EPAD - E
    src = jnp.concatenate(
        [edge_index[0], jnp.full((pad,), DUMMY, jnp.int32)]
    ).reshape(NC, NS, C, CHUNK)
    dst = jnp.concatenate(
        [edge_index[1], jnp.full((pad,), DUMMY, jnp.int32)]
    ).reshape(NC, NS, C, CHUNK)

    degp = _sc_degree(dst)
    g1, dinv16 = _tc_pre(xp, W1, degp[0], degp[1])

    s1 = _sc_propagate(g1, src, dst)
    g2 = _tc_mid(s1[0], s1[1], g1, dinv16, b1.reshape(1, D),
                 bn1_g.reshape(1, D), bn1_b.reshape(1, D), W2)

    s2 = _sc_propagate(g2, src, dst)
    g3 = _tc_mid(s2[0], s2[1], g2, dinv16, b2.reshape(1, D),
                 bn2_g.reshape(1, D), bn2_b.reshape(1, D), W3)

    s3 = _sc_propagate(g3, src, dst)
    node_embeddings, graph_embedding = _tc_post(
        s3[0], s3[1], g3, dinv16, b3.reshape(1, D),
        ln_g.reshape(1, D), ln_b.reshape(1, D))

    return (node_embeddings, graph_embedding)
